# Initial kernel scaffold; baseline (speedup 1.0000x reference)
import jax
import jax.numpy as jnp
from jax.experimental import pallas as pl


def kernel(x, edge_index, Wl1, bl1, Wr1, Wl2, bl2, Wr2):
    raise NotImplementedError("write your pallas kernel here")


# speedup vs baseline: 7.2092x; 7.2092x over previous
import functools

import jax
import jax.numpy as jnp
from jax import lax
from jax.experimental import pallas as pl
from jax.experimental.pallas import tpu as pltpu
from jax.experimental.pallas import tpu_sc as plsc

N = 10000
E = 320000
D = 128

NC = 2
NS = 16
NW = NC * NS
E_W = E // NW
CHUNK = 80
NCHUNK = E_W // CHUNK
ROWS_T = 624
ROWS_TAIL = N - ROWS_T * NS
ZB = 800


def _make_sc_agg(compute_deg: bool):
    out_type = [jax.ShapeDtypeStruct((NC, N, D), jnp.float32)]
    if compute_deg:
        out_type.append(jax.ShapeDtypeStruct((NC, N), jnp.float32))

    scratch = [
        pltpu.VMEM((NCHUNK, CHUNK), jnp.int32),
        pltpu.VMEM((NCHUNK, CHUNK), jnp.int32),
        pltpu.VMEM((CHUNK, D), jnp.float32),
        pltpu.VMEM((CHUNK,), jnp.float32),
        pltpu.VMEM((ZB,), jnp.float32),
        pltpu.VMEM_SHARED((N, D), jnp.float32),
        pltpu.VMEM_SHARED((N,), jnp.float32),
        pltpu.SemaphoreType.DMA,
    ]

    mesh = plsc.VectorSubcoreMesh(core_axis_name="c", subcore_axis_name="s")

    def body(x_hbm, src_hbm, dst_hbm, *rest):
        if compute_deg:
            agg_out, deg_out = rest[0], rest[1]
            scr = rest[2:]
        else:
            agg_out = rest[0]
            deg_out = None
            scr = rest[1:]
        src_v, dst_v, rows_v, ones_v, z1_v, agg_s, deg_s, sem = scr

        c = lax.axis_index("c")
        s = lax.axis_index("s")
        wid = s * NC + c

        z16 = jnp.zeros((16,), jnp.float32)

        def zrow(i, carry):
            for j in range(D // 16):
                rows_v[i, pl.ds(j * 16, 16)] = z16
            return carry

        lax.fori_loop(0, CHUNK, zrow, 0)

        r0 = s * ROWS_T
        n_full = ROWS_T // CHUNK
        rem = ROWS_T - n_full * CHUNK
        for t in range(n_full):
            pltpu.sync_copy(rows_v, agg_s.at[pl.ds(r0 + t * CHUNK, CHUNK)])
        if rem:
            pltpu.sync_copy(rows_v.at[pl.ds(0, rem)],
                            agg_s.at[pl.ds(r0 + n_full * CHUNK, rem)])

        @pl.when(s == 0)
        def _():
            pltpu.sync_copy(rows_v.at[pl.ds(0, ROWS_TAIL)],
                            agg_s.at[pl.ds(ROWS_T * NS, ROWS_TAIL)])

        if compute_deg:
            def zz(i, carry):
                z1_v[pl.ds(i * 16, 16)] = z16
                return carry
            lax.fori_loop(0, ZB // 16, zz, 0)

            one16 = jnp.ones((16,), jnp.float32)
            for j in range(CHUNK // 16):
                ones_v[pl.ds(j * 16, 16)] = one16

            @pl.when(s == 0)
            def _():
                nf = N // ZB
                for t in range(nf):
                    pltpu.sync_copy(z1_v, deg_s.at[pl.ds(t * ZB, ZB)])
                drem = N - nf * ZB
                if drem:
                    pltpu.sync_copy(z1_v.at[pl.ds(0, drem)],
                                    deg_s.at[pl.ds(nf * ZB, drem)])

        pltpu.sync_copy(src_hbm.at[wid], src_v)
        pltpu.sync_copy(dst_hbm.at[wid], dst_v)

        plsc.subcore_barrier()

        def step(k, carry):
            pltpu.async_copy(x_hbm.at[src_v.at[k]], rows_v, sem).wait()
            pltpu.sync_copy(rows_v, agg_s.at[dst_v.at[k]], add=True)
            if compute_deg:
                pltpu.sync_copy(ones_v, deg_s.at[dst_v.at[k]], add=True)
            return carry

        lax.fori_loop(0, NCHUNK, step, 0)

        plsc.subcore_barrier()

        pltpu.sync_copy(agg_s.at[pl.ds(r0, ROWS_T)],
                        agg_out.at[c, pl.ds(r0, ROWS_T)])

        @pl.when(s == 0)
        def _():
            pltpu.sync_copy(agg_s.at[pl.ds(ROWS_T * NS, ROWS_TAIL)],
                            agg_out.at[c, pl.ds(ROWS_T * NS, ROWS_TAIL)])

        if compute_deg:
            @pl.when(s == 0)
            def _():
                pltpu.sync_copy(deg_s, deg_out.at[c])

    return pl.kernel(body, out_type=out_type, scratch_types=scratch, mesh=mesh)


_sc_agg_deg = _make_sc_agg(True)
_sc_agg = _make_sc_agg(False)


RB = 1000
NB = N // RB


def _tc_hidden_body(aggp, degp, x, wl, bl, wr, o):
    agg = aggp[0] + aggp[1]
    deg = jnp.maximum(degp[0] + degp[1], 1.0)
    mean = agg / deg
    z = (lax.dot_general(mean, wl[...], (((1,), (1,)), ((), ())),
                         preferred_element_type=jnp.float32)
         + lax.dot_general(x[...], wr[...], (((1,), (1,)), ((), ())),
                           preferred_element_type=jnp.float32)
         + bl[...])
    o[...] = jnp.maximum(z, 0.0)


def _tc_final_body(aggp, degp, x, wl, bl, wr, o):
    agg = aggp[0] + aggp[1]
    deg = jnp.maximum(degp[0] + degp[1], 1.0)
    mean = agg / deg
    z = (lax.dot_general(mean, wl[...], (((1,), (1,)), ((), ())),
                         preferred_element_type=jnp.float32)
         + lax.dot_general(x[...], wr[...], (((1,), (1,)), ((), ())),
                           preferred_element_type=jnp.float32)
         + bl[...])
    m = jnp.max(z, axis=-1, keepdims=True)
    lse = jnp.log(jnp.sum(jnp.exp(z - m), axis=-1, keepdims=True)) + m
    o[...] = z - lse


def _tc_layer(body, aggp, degp, x, wl, bl, wr):
    return pl.pallas_call(
        body,
        grid=(NB,),
        in_specs=[
            pl.BlockSpec((NC, RB, D), lambda i: (0, i, 0)),
            pl.BlockSpec((NC, RB, 1), lambda i: (0, i, 0)),
            pl.BlockSpec((RB, D), lambda i: (i, 0)),
            pl.BlockSpec((D, D), lambda i: (0, 0)),
            pl.BlockSpec((1, D), lambda i: (0, 0)),
            pl.BlockSpec((D, D), lambda i: (0, 0)),
        ],
        out_specs=pl.BlockSpec((RB, D), lambda i: (i, 0)),
        out_shape=jax.ShapeDtypeStruct((N, D), jnp.float32),
    )(aggp, degp, x, wl, bl, wr)


def kernel(x, edge_index, Wl1, bl1, Wr1, Wl2, bl2, Wr2):
    src = edge_index[0].astype(jnp.int32).reshape(NW, NCHUNK, CHUNK)
    dst = edge_index[1].astype(jnp.int32).reshape(NW, NCHUNK, CHUNK)

    aggp1, degp = _sc_agg_deg(x, src, dst)
    degp3 = degp[:, :, None]
    h = _tc_layer(_tc_hidden_body, aggp1, degp3, x,
                  Wl1, bl1.reshape(1, D), Wr1)
    (aggp2,) = _sc_agg(h, src, dst)
    out = _tc_layer(_tc_final_body, aggp2, degp3, h,
                    Wl2, bl2.reshape(1, D), Wr2)
    return out


# speedup vs baseline: 12.1040x; 1.6790x over previous
import functools

import jax
import jax.numpy as jnp
from jax import lax
from jax.experimental import pallas as pl
from jax.experimental.pallas import tpu as pltpu
from jax.experimental.pallas import tpu_sc as plsc

N = 10000
E = 320000
D = 128

NC = 2
NS = 16
NW = NC * NS
E_W = E // NW
CHUNK = 128
NCHUNK = 78
NWIN = 3
WCH = NCHUNK // NWIN
TAIL = E_W - NCHUNK * CHUNK
ROWS_T = 624
ROWS_TAIL = N - ROWS_T * NS
ZB = 800


def _make_sc_agg(compute_deg: bool):
    out_type = [jax.ShapeDtypeStruct((NC, N, D), jnp.float32)]
    if compute_deg:
        out_type.append(jax.ShapeDtypeStruct((NC, N), jnp.float32))

    scratch = [
        pltpu.VMEM((WCH, CHUNK), jnp.int32),
        pltpu.VMEM((WCH, CHUNK), jnp.int32),
        pltpu.VMEM((1, TAIL), jnp.int32),
        pltpu.VMEM((1, TAIL), jnp.int32),
        pltpu.VMEM((CHUNK, D), jnp.float32),
        pltpu.VMEM((CHUNK, D), jnp.float32),
        pltpu.VMEM((CHUNK,), jnp.float32),
        pltpu.VMEM((ZB,), jnp.float32),
        pltpu.VMEM_SHARED((N, D), jnp.float32),
        pltpu.VMEM_SHARED((N,), jnp.float32),
        pltpu.SemaphoreType.DMA,
        pltpu.SemaphoreType.DMA,
    ]

    mesh = plsc.VectorSubcoreMesh(core_axis_name="c", subcore_axis_name="s")

    def body(x_hbm, srcm_hbm, dstm_hbm, srct_hbm, dstt_hbm, *rest):
        if compute_deg:
            agg_out, deg_out = rest[0], rest[1]
            scr = rest[2:]
        else:
            agg_out = rest[0]
            deg_out = None
            scr = rest[1:]
        (src_v, dst_v, srct_v, dstt_v, rows0, rows1, ones_v, z1_v,
         agg_s, deg_s, sem0, sem1) = scr

        c = lax.axis_index("c")
        s = lax.axis_index("s")
        wid = s * NC + c

        z16 = jnp.zeros((16,), jnp.float32)

        def zrow(i, carry):
            for j in range(D // 16):
                rows0[i, pl.ds(j * 16, 16)] = z16
            return carry

        lax.fori_loop(0, CHUNK, zrow, 0)

        r0 = s * ROWS_T
        n_full = ROWS_T // CHUNK
        rem = ROWS_T - n_full * CHUNK
        for t in range(n_full):
            pltpu.sync_copy(rows0, agg_s.at[pl.ds(r0 + t * CHUNK, CHUNK)])
        if rem:
            pltpu.sync_copy(rows0.at[pl.ds(0, rem)],
                            agg_s.at[pl.ds(r0 + n_full * CHUNK, rem)])

        @pl.when(s == 0)
        def _():
            pltpu.sync_copy(rows0.at[pl.ds(0, ROWS_TAIL)],
                            agg_s.at[pl.ds(ROWS_T * NS, ROWS_TAIL)])

        if compute_deg:
            def zz(i, carry):
                z1_v[pl.ds(i * 16, 16)] = z16
                return carry
            lax.fori_loop(0, ZB // 16, zz, 0)

            one16 = jnp.ones((16,), jnp.float32)
            for j in range(CHUNK // 16):
                ones_v[pl.ds(j * 16, 16)] = one16

            @pl.when(s == 0)
            def _():
                nf = N // ZB
                for t in range(nf):
                    pltpu.sync_copy(z1_v, deg_s.at[pl.ds(t * ZB, ZB)])
                drem = N - nf * ZB
                if drem:
                    pltpu.sync_copy(z1_v.at[pl.ds(0, drem)],
                                    deg_s.at[pl.ds(nf * ZB, drem)])

        pltpu.sync_copy(srct_hbm.at[wid], srct_v)
        pltpu.sync_copy(dstt_hbm.at[wid], dstt_v)

        plsc.subcore_barrier()

        def gather(k, buf, sem):
            return pltpu.async_copy(x_hbm.at[src_v.at[k]], buf, sem)

        def scat(k, buf):
            pltpu.sync_copy(buf, agg_s.at[dst_v.at[k]], add=True)
            if compute_deg:
                pltpu.sync_copy(ones_v, deg_s.at[dst_v.at[k]], add=True)

        def step(g, carry):
            k0 = 2 * g
            gather(k0 + 1, rows1, sem1)
            pltpu.make_async_copy(x_hbm.at[src_v.at[k0]], rows0, sem0).wait()
            scat(k0, rows0)

            @pl.when(k0 + 2 < WCH)
            def _():
                gather(k0 + 2, rows0, sem0)

            pltpu.make_async_copy(x_hbm.at[src_v.at[k0 + 1]], rows1,
                                  sem1).wait()
            scat(k0 + 1, rows1)
            return carry

        for w in range(NWIN):
            pltpu.sync_copy(srcm_hbm.at[wid, w], src_v)
            pltpu.sync_copy(dstm_hbm.at[wid, w], dst_v)
            gather(0, rows0, sem0)
            lax.fori_loop(0, WCH // 2, step, 0)

        pltpu.async_copy(x_hbm.at[srct_v.at[0]],
                         rows0.at[pl.ds(0, TAIL)], sem0).wait()
        pltpu.sync_copy(rows0.at[pl.ds(0, TAIL)],
                        agg_s.at[dstt_v.at[0]], add=True)
        if compute_deg:
            pltpu.sync_copy(ones_v.at[pl.ds(0, TAIL)],
                            deg_s.at[dstt_v.at[0]], add=True)

        plsc.subcore_barrier()

        pltpu.sync_copy(agg_s.at[pl.ds(r0, ROWS_T)],
                        agg_out.at[c, pl.ds(r0, ROWS_T)])

        @pl.when(s == 0)
        def _():
            pltpu.sync_copy(agg_s.at[pl.ds(ROWS_T * NS, ROWS_TAIL)],
                            agg_out.at[c, pl.ds(ROWS_T * NS, ROWS_TAIL)])

        if compute_deg:
            @pl.when(s == 0)
            def _():
                pltpu.sync_copy(deg_s, deg_out.at[c])

    return pl.kernel(body, out_type=out_type, scratch_types=scratch, mesh=mesh)


_sc_agg_deg = _make_sc_agg(True)
_sc_agg = _make_sc_agg(False)


RB = 1000
NB = N // RB


def _tc_hidden_body(aggp, degp, x, wl, bl, wr, o):
    agg = aggp[0] + aggp[1]
    deg = jnp.maximum(degp[0] + degp[1], 1.0)
    mean = agg / deg
    z = (lax.dot_general(mean, wl[...], (((1,), (1,)), ((), ())),
                         preferred_element_type=jnp.float32)
         + lax.dot_general(x[...], wr[...], (((1,), (1,)), ((), ())),
                           preferred_element_type=jnp.float32)
         + bl[...])
    o[...] = jnp.maximum(z, 0.0)


def _tc_final_body(aggp, degp, x, wl, bl, wr, o):
    agg = aggp[0] + aggp[1]
    deg = jnp.maximum(degp[0] + degp[1], 1.0)
    mean = agg / deg
    z = (lax.dot_general(mean, wl[...], (((1,), (1,)), ((), ())),
                         preferred_element_type=jnp.float32)
         + lax.dot_general(x[...], wr[...], (((1,), (1,)), ((), ())),
                           preferred_element_type=jnp.float32)
         + bl[...])
    m = jnp.max(z, axis=-1, keepdims=True)
    lse = jnp.log(jnp.sum(jnp.exp(z - m), axis=-1, keepdims=True)) + m
    o[...] = z - lse


def _tc_layer(body, aggp, degp, x, wl, bl, wr):
    return pl.pallas_call(
        body,
        grid=(NB,),
        in_specs=[
            pl.BlockSpec((NC, RB, D), lambda i: (0, i, 0)),
            pl.BlockSpec((NC, RB, 1), lambda i: (0, i, 0)),
            pl.BlockSpec((RB, D), lambda i: (i, 0)),
            pl.BlockSpec((D, D), lambda i: (0, 0)),
            pl.BlockSpec((1, D), lambda i: (0, 0)),
            pl.BlockSpec((D, D), lambda i: (0, 0)),
        ],
        out_specs=pl.BlockSpec((RB, D), lambda i: (i, 0)),
        out_shape=jax.ShapeDtypeStruct((N, D), jnp.float32),
    )(aggp, degp, x, wl, bl, wr)


def kernel(x, edge_index, Wl1, bl1, Wr1, Wl2, bl2, Wr2):
    ei = edge_index.astype(jnp.int32).reshape(2, NW, E_W)
    main = ei[:, :, :NCHUNK * CHUNK].reshape(2, NW, NWIN, WCH, CHUNK)
    tail = ei[:, :, NCHUNK * CHUNK:].reshape(2, NW, 1, TAIL)
    srcm, dstm = main[0], main[1]
    srct, dstt = tail[0], tail[1]

    aggp1, degp = _sc_agg_deg(x, srcm, dstm, srct, dstt)
    degp3 = degp[:, :, None]
    h = _tc_layer(_tc_hidden_body, aggp1, degp3, x,
                  Wl1, bl1.reshape(1, D), Wr1)
    (aggp2,) = _sc_agg(h, srcm, dstm, srct, dstt)
    out = _tc_layer(_tc_final_body, aggp2, degp3, h,
                    Wl2, bl2.reshape(1, D), Wr2)
    return out


# speedup vs baseline: 12.4818x; 1.0312x over previous
import functools

import jax
import jax.numpy as jnp
from jax import lax
from jax.experimental import pallas as pl
from jax.experimental.pallas import tpu as pltpu
from jax.experimental.pallas import tpu_sc as plsc

N = 10000
E = 320000
D = 128

NC = 2
NS = 16
NW = NC * NS
E_W = E // NW
CHUNK = 128
NCHUNK = 78
NWIN = 3
WCH = NCHUNK // NWIN
TAIL = E_W - NCHUNK * CHUNK
ROWS_T = 624
ROWS_TAIL = N - ROWS_T * NS
ZB = 800


def _make_sc_agg(compute_deg: bool):
    out_type = [jax.ShapeDtypeStruct((NC, N, D), jnp.float32)]
    if compute_deg:
        out_type.append(jax.ShapeDtypeStruct((NC, N), jnp.float32))

    scratch = [
        pltpu.VMEM((WCH, CHUNK), jnp.int32),
        pltpu.VMEM((WCH, CHUNK), jnp.int32),
        pltpu.VMEM((1, TAIL), jnp.int32),
        pltpu.VMEM((1, TAIL), jnp.int32),
        pltpu.VMEM((CHUNK, D), jnp.float32),
        pltpu.VMEM((CHUNK, D), jnp.float32),
        pltpu.VMEM((CHUNK,), jnp.float32),
        pltpu.VMEM((ZB,), jnp.float32),
        pltpu.VMEM_SHARED((N, D), jnp.float32),
        pltpu.VMEM_SHARED((N,), jnp.float32),
        pltpu.SemaphoreType.DMA,
        pltpu.SemaphoreType.DMA,
    ]

    mesh = plsc.VectorSubcoreMesh(core_axis_name="c", subcore_axis_name="s")

    def body(x_hbm, srcm_hbm, dstm_hbm, srct_hbm, dstt_hbm, *rest):
        if compute_deg:
            agg_out, deg_out = rest[0], rest[1]
            scr = rest[2:]
        else:
            agg_out = rest[0]
            deg_out = None
            scr = rest[1:]
        (src_v, dst_v, srct_v, dstt_v, rows0, rows1, ones_v, z1_v,
         agg_s, deg_s, sem0, sem1) = scr

        c = lax.axis_index("c")
        s = lax.axis_index("s")
        wid = s * NC + c

        z16 = jnp.zeros((16,), jnp.float32)

        pltpu.sync_copy(srcm_hbm.at[wid, 0], src_v)
        pltpu.sync_copy(dstm_hbm.at[wid, 0], dst_v)
        pltpu.sync_copy(srct_hbm.at[wid], srct_v)
        pltpu.sync_copy(dstt_hbm.at[wid], dstt_v)
        pltpu.async_copy(x_hbm.at[src_v.at[0]], rows0, sem0)

        def zrow(i, carry):
            for j in range(D // 16):
                rows1[i, pl.ds(j * 16, 16)] = z16
            return carry

        lax.fori_loop(0, CHUNK, zrow, 0)

        r0 = s * ROWS_T
        n_full = ROWS_T // CHUNK
        rem = ROWS_T - n_full * CHUNK
        for t in range(n_full):
            pltpu.sync_copy(rows1, agg_s.at[pl.ds(r0 + t * CHUNK, CHUNK)])
        if rem:
            pltpu.sync_copy(rows1.at[pl.ds(0, rem)],
                            agg_s.at[pl.ds(r0 + n_full * CHUNK, rem)])

        @pl.when(s == 0)
        def _():
            pltpu.sync_copy(rows1.at[pl.ds(0, ROWS_TAIL)],
                            agg_s.at[pl.ds(ROWS_T * NS, ROWS_TAIL)])

        if compute_deg:
            def zz(i, carry):
                z1_v[pl.ds(i * 16, 16)] = z16
                return carry
            lax.fori_loop(0, ZB // 16, zz, 0)

            one16 = jnp.ones((16,), jnp.float32)
            for j in range(CHUNK // 16):
                ones_v[pl.ds(j * 16, 16)] = one16

            @pl.when(s == 0)
            def _():
                nf = N // ZB
                for t in range(nf):
                    pltpu.sync_copy(z1_v, deg_s.at[pl.ds(t * ZB, ZB)])
                drem = N - nf * ZB
                if drem:
                    pltpu.sync_copy(z1_v.at[pl.ds(0, drem)],
                                    deg_s.at[pl.ds(nf * ZB, drem)])

        plsc.subcore_barrier()

        def gather(k, buf, sem):
            return pltpu.async_copy(x_hbm.at[src_v.at[k]], buf, sem)

        def scat(k, buf):
            pltpu.sync_copy(buf, agg_s.at[dst_v.at[k]], add=True)
            if compute_deg:
                pltpu.sync_copy(ones_v, deg_s.at[dst_v.at[k]], add=True)

        def step(g, carry):
            k0 = 2 * g
            gather(k0 + 1, rows1, sem1)
            pltpu.make_async_copy(x_hbm.at[src_v.at[k0]], rows0, sem0).wait()
            scat(k0, rows0)

            @pl.when(k0 + 2 < WCH)
            def _():
                gather(k0 + 2, rows0, sem0)

            pltpu.make_async_copy(x_hbm.at[src_v.at[k0 + 1]], rows1,
                                  sem1).wait()
            scat(k0 + 1, rows1)
            return carry

        for w in range(NWIN):
            if w > 0:
                pltpu.sync_copy(srcm_hbm.at[wid, w], src_v)
                pltpu.sync_copy(dstm_hbm.at[wid, w], dst_v)
                gather(0, rows0, sem0)
            lax.fori_loop(0, WCH // 2, step, 0)

        pltpu.async_copy(x_hbm.at[srct_v.at[0]],
                         rows0.at[pl.ds(0, TAIL)], sem0).wait()
        pltpu.sync_copy(rows0.at[pl.ds(0, TAIL)],
                        agg_s.at[dstt_v.at[0]], add=True)
        if compute_deg:
            pltpu.sync_copy(ones_v.at[pl.ds(0, TAIL)],
                            deg_s.at[dstt_v.at[0]], add=True)

        plsc.subcore_barrier()

        pltpu.sync_copy(agg_s.at[pl.ds(r0, ROWS_T)],
                        agg_out.at[c, pl.ds(r0, ROWS_T)])

        @pl.when(s == 0)
        def _():
            pltpu.sync_copy(agg_s.at[pl.ds(ROWS_T * NS, ROWS_TAIL)],
                            agg_out.at[c, pl.ds(ROWS_T * NS, ROWS_TAIL)])

        if compute_deg:
            @pl.when(s == 0)
            def _():
                pltpu.sync_copy(deg_s, deg_out.at[c])

    return pl.kernel(body, out_type=out_type, scratch_types=scratch, mesh=mesh)


_sc_agg_deg = _make_sc_agg(True)
_sc_agg = _make_sc_agg(False)


RB = 2000
NB = N // RB


def _tc_hidden_body(aggp, degp, x, wl, bl, wr, o):
    agg = aggp[0] + aggp[1]
    deg = jnp.maximum(degp[0] + degp[1], 1.0)
    mean = agg / deg
    z = (lax.dot_general(mean, wl[...], (((1,), (1,)), ((), ())),
                         preferred_element_type=jnp.float32)
         + lax.dot_general(x[...], wr[...], (((1,), (1,)), ((), ())),
                           preferred_element_type=jnp.float32)
         + bl[...])
    o[...] = jnp.maximum(z, 0.0)


def _tc_final_body(aggp, degp, x, wl, bl, wr, o):
    agg = aggp[0] + aggp[1]
    deg = jnp.maximum(degp[0] + degp[1], 1.0)
    mean = agg / deg
    z = (lax.dot_general(mean, wl[...], (((1,), (1,)), ((), ())),
                         preferred_element_type=jnp.float32)
         + lax.dot_general(x[...], wr[...], (((1,), (1,)), ((), ())),
                           preferred_element_type=jnp.float32)
         + bl[...])
    m = jnp.max(z, axis=-1, keepdims=True)
    lse = jnp.log(jnp.sum(jnp.exp(z - m), axis=-1, keepdims=True)) + m
    o[...] = z - lse


def _tc_layer(body, aggp, degp, x, wl, bl, wr):
    return pl.pallas_call(
        body,
        grid=(NB,),
        in_specs=[
            pl.BlockSpec((NC, RB, D), lambda i: (0, i, 0)),
            pl.BlockSpec((NC, RB, 1), lambda i: (0, i, 0)),
            pl.BlockSpec((RB, D), lambda i: (i, 0)),
            pl.BlockSpec((D, D), lambda i: (0, 0)),
            pl.BlockSpec((1, D), lambda i: (0, 0)),
            pl.BlockSpec((D, D), lambda i: (0, 0)),
        ],
        out_specs=pl.BlockSpec((RB, D), lambda i: (i, 0)),
        out_shape=jax.ShapeDtypeStruct((N, D), jnp.float32),
    )(aggp, degp, x, wl, bl, wr)


def kernel(x, edge_index, Wl1, bl1, Wr1, Wl2, bl2, Wr2):
    ei = edge_index.astype(jnp.int32).reshape(2, NW, E_W)
    main = ei[:, :, :NCHUNK * CHUNK].reshape(2, NW, NWIN, WCH, CHUNK)
    tail = ei[:, :, NCHUNK * CHUNK:].reshape(2, NW, 1, TAIL)
    srcm, dstm = main[0], main[1]
    srct, dstt = tail[0], tail[1]

    aggp1, degp = _sc_agg_deg(x, srcm, dstm, srct, dstt)
    degp3 = degp[:, :, None]
    h = _tc_layer(_tc_hidden_body, aggp1, degp3, x,
                  Wl1, bl1.reshape(1, D), Wr1)
    (aggp2,) = _sc_agg(h, srcm, dstm, srct, dstt)
    out = _tc_layer(_tc_final_body, aggp2, degp3, h,
                    Wl2, bl2.reshape(1, D), Wr2)
    return out


# speedup vs baseline: 12.6463x; 1.0132x over previous
import functools

import jax
import jax.numpy as jnp
from jax import lax
from jax.experimental import pallas as pl
from jax.experimental.pallas import tpu as pltpu
from jax.experimental.pallas import tpu_sc as plsc

N = 10000
E = 320000
D = 128

NC = 2
NS = 16
NW = NC * NS
E_W = E // NW
CHUNK = 128
NCHUNK = 78
NWIN = 3
WCH = NCHUNK // NWIN
TAIL = E_W - NCHUNK * CHUNK
ROWS_T = 624
ROWS_TAIL = N - ROWS_T * NS
ZB = 800


def _make_sc_agg(compute_deg: bool):
    out_type = [jax.ShapeDtypeStruct((NC, N, D), jnp.float32)]
    if compute_deg:
        out_type.append(jax.ShapeDtypeStruct((NC, N), jnp.float32))

    scratch = [
        pltpu.VMEM((2, WCH, CHUNK), jnp.int32),
        pltpu.VMEM((2, WCH, CHUNK), jnp.int32),
        pltpu.VMEM((1, TAIL), jnp.int32),
        pltpu.VMEM((1, TAIL), jnp.int32),
        pltpu.VMEM((CHUNK, D), jnp.float32),
        pltpu.VMEM((CHUNK, D), jnp.float32),
        pltpu.VMEM((CHUNK,), jnp.float32),
        pltpu.VMEM((ZB,), jnp.float32),
        pltpu.VMEM_SHARED((N, D), jnp.float32),
        pltpu.VMEM_SHARED((N,), jnp.float32),
        pltpu.SemaphoreType.DMA,
        pltpu.SemaphoreType.DMA,
        pltpu.SemaphoreType.DMA,
    ]

    mesh = plsc.VectorSubcoreMesh(core_axis_name="c", subcore_axis_name="s")

    def body(x_hbm, srcm_hbm, dstm_hbm, srct_hbm, dstt_hbm, *rest):
        if compute_deg:
            agg_out, deg_out = rest[0], rest[1]
            scr = rest[2:]
        else:
            agg_out = rest[0]
            deg_out = None
            scr = rest[1:]
        (src_v, dst_v, srct_v, dstt_v, rows0, rows1, ones_v, z1_v,
         agg_s, deg_s, sem0, sem1, semw) = scr

        c = lax.axis_index("c")
        s = lax.axis_index("s")
        wid = s * NC + c

        z16 = jnp.zeros((16,), jnp.float32)

        pltpu.sync_copy(srcm_hbm.at[wid, 0], src_v.at[0])
        pltpu.sync_copy(dstm_hbm.at[wid, 0], dst_v.at[0])
        pltpu.sync_copy(srct_hbm.at[wid], srct_v)
        pltpu.sync_copy(dstt_hbm.at[wid], dstt_v)
        pltpu.async_copy(x_hbm.at[src_v.at[0, 0]], rows0, sem0)

        def zrow(i, carry):
            for j in range(D // 16):
                rows1[i, pl.ds(j * 16, 16)] = z16
            return carry

        lax.fori_loop(0, CHUNK, zrow, 0)

        r0 = s * ROWS_T
        n_full = ROWS_T // CHUNK
        rem = ROWS_T - n_full * CHUNK
        for t in range(n_full):
            pltpu.sync_copy(rows1, agg_s.at[pl.ds(r0 + t * CHUNK, CHUNK)])
        if rem:
            pltpu.sync_copy(rows1.at[pl.ds(0, rem)],
                            agg_s.at[pl.ds(r0 + n_full * CHUNK, rem)])

        @pl.when(s == 0)
        def _():
            pltpu.sync_copy(rows1.at[pl.ds(0, ROWS_TAIL)],
                            agg_s.at[pl.ds(ROWS_T * NS, ROWS_TAIL)])

        if compute_deg:
            def zz(i, carry):
                z1_v[pl.ds(i * 16, 16)] = z16
                return carry
            lax.fori_loop(0, ZB // 16, zz, 0)

            one16 = jnp.ones((16,), jnp.float32)
            for j in range(CHUNK // 16):
                ones_v[pl.ds(j * 16, 16)] = one16

            @pl.when(s == 0)
            def _():
                nf = N // ZB
                for t in range(nf):
                    pltpu.sync_copy(z1_v, deg_s.at[pl.ds(t * ZB, ZB)])
                drem = N - nf * ZB
                if drem:
                    pltpu.sync_copy(z1_v.at[pl.ds(0, drem)],
                                    deg_s.at[pl.ds(nf * ZB, drem)])

        plsc.subcore_barrier()

        def gather(b, k, buf, sem):
            return pltpu.async_copy(x_hbm.at[src_v.at[b, k]], buf, sem)

        def scat(b, k, buf):
            pltpu.sync_copy(buf, agg_s.at[dst_v.at[b, k]], add=True)
            if compute_deg:
                pltpu.sync_copy(ones_v, deg_s.at[dst_v.at[b, k]], add=True)

        def make_step(b):
            def step(g, carry):
                k0 = 2 * g
                gather(b, k0 + 1, rows1, sem1)
                pltpu.make_async_copy(x_hbm.at[src_v.at[b, k0]], rows0,
                                      sem0).wait()
                scat(b, k0, rows0)

                @pl.when(k0 + 2 < WCH)
                def _():
                    gather(b, k0 + 2, rows0, sem0)

                pltpu.make_async_copy(x_hbm.at[src_v.at[b, k0 + 1]], rows1,
                                      sem1).wait()
                scat(b, k0 + 1, rows1)
                return carry
            return step

        for w in range(NWIN):
            b = w % 2
            if w + 1 < NWIN:
                nb = (w + 1) % 2
                pltpu.async_copy(srcm_hbm.at[wid, w + 1], src_v.at[nb], semw)
                pltpu.async_copy(dstm_hbm.at[wid, w + 1], dst_v.at[nb], semw)
            lax.fori_loop(0, WCH // 2, make_step(b), 0)
            if w + 1 < NWIN:
                nb = (w + 1) % 2
                pltpu.make_async_copy(srcm_hbm.at[wid, w + 1], src_v.at[nb],
                                      semw).wait()
                pltpu.make_async_copy(dstm_hbm.at[wid, w + 1], dst_v.at[nb],
                                      semw).wait()
                gather(nb, 0, rows0, sem0)

        pltpu.async_copy(x_hbm.at[srct_v.at[0]],
                         rows0.at[pl.ds(0, TAIL)], sem0).wait()
        pltpu.sync_copy(rows0.at[pl.ds(0, TAIL)],
                        agg_s.at[dstt_v.at[0]], add=True)
        if compute_deg:
            pltpu.sync_copy(ones_v.at[pl.ds(0, TAIL)],
                            deg_s.at[dstt_v.at[0]], add=True)

        plsc.subcore_barrier()

        pltpu.sync_copy(agg_s.at[pl.ds(r0, ROWS_T)],
                        agg_out.at[c, pl.ds(r0, ROWS_T)])

        @pl.when(s == 0)
        def _():
            pltpu.sync_copy(agg_s.at[pl.ds(ROWS_T * NS, ROWS_TAIL)],
                            agg_out.at[c, pl.ds(ROWS_T * NS, ROWS_TAIL)])

        if compute_deg:
            @pl.when(s == 0)
            def _():
                pltpu.sync_copy(deg_s, deg_out.at[c])

    return pl.kernel(body, out_type=out_type, scratch_types=scratch, mesh=mesh)


_sc_agg_deg = _make_sc_agg(True)
_sc_agg = _make_sc_agg(False)


RB = 2000
NB = N // RB


def _tc_hidden_body(aggp, degp, x, wl, bl, wr, o):
    agg = aggp[0] + aggp[1]
    deg = jnp.maximum(degp[0] + degp[1], 1.0)
    mean = agg / deg
    z = (lax.dot_general(mean, wl[...], (((1,), (1,)), ((), ())),
                         preferred_element_type=jnp.float32)
         + lax.dot_general(x[...], wr[...], (((1,), (1,)), ((), ())),
                           preferred_element_type=jnp.float32)
         + bl[...])
    o[...] = jnp.maximum(z, 0.0)


def _tc_final_body(aggp, degp, x, wl, bl, wr, o):
    agg = aggp[0] + aggp[1]
    deg = jnp.maximum(degp[0] + degp[1], 1.0)
    mean = agg / deg
    z = (lax.dot_general(mean, wl[...], (((1,), (1,)), ((), ())),
                         preferred_element_type=jnp.float32)
         + lax.dot_general(x[...], wr[...], (((1,), (1,)), ((), ())),
                           preferred_element_type=jnp.float32)
         + bl[...])
    m = jnp.max(z, axis=-1, keepdims=True)
    lse = jnp.log(jnp.sum(jnp.exp(z - m), axis=-1, keepdims=True)) + m
    o[...] = z - lse


def _tc_layer(body, aggp, degp, x, wl, bl, wr):
    return pl.pallas_call(
        body,
        grid=(NB,),
        in_specs=[
            pl.BlockSpec((NC, RB, D), lambda i: (0, i, 0)),
            pl.BlockSpec((NC, RB, 1), lambda i: (0, i, 0)),
            pl.BlockSpec((RB, D), lambda i: (i, 0)),
            pl.BlockSpec((D, D), lambda i: (0, 0)),
            pl.BlockSpec((1, D), lambda i: (0, 0)),
            pl.BlockSpec((D, D), lambda i: (0, 0)),
        ],
        out_specs=pl.BlockSpec((RB, D), lambda i: (i, 0)),
        out_shape=jax.ShapeDtypeStruct((N, D), jnp.float32),
    )(aggp, degp, x, wl, bl, wr)


def kernel(x, edge_index, Wl1, bl1, Wr1, Wl2, bl2, Wr2):
    ei = edge_index.astype(jnp.int32).reshape(2, NW, E_W)
    main = ei[:, :, :NCHUNK * CHUNK].reshape(2, NW, NWIN, WCH, CHUNK)
    tail = ei[:, :, NCHUNK * CHUNK:].reshape(2, NW, 1, TAIL)
    srcm, dstm = main[0], main[1]
    srct, dstt = tail[0], tail[1]

    aggp1, degp = _sc_agg_deg(x, srcm, dstm, srct, dstt)
    degp3 = degp[:, :, None]
    h = _tc_layer(_tc_hidden_body, aggp1, degp3, x,
                  Wl1, bl1.reshape(1, D), Wr1)
    (aggp2,) = _sc_agg(h, srcm, dstm, srct, dstt)
    out = _tc_layer(_tc_final_body, aggp2, degp3, h,
                    Wl2, bl2.reshape(1, D), Wr2)
    return out


# speedup vs baseline: 13.0613x; 1.0328x over previous
import functools

import jax
import jax.numpy as jnp
from jax import lax
from jax.experimental import pallas as pl
from jax.experimental.pallas import tpu as pltpu
from jax.experimental.pallas import tpu_sc as plsc

N = 10000
E = 320000
D = 128

NC = 2
NS = 16
NW = NC * NS
E_W = E // NW
CHUNK = 128
NCHUNK = 78
NWIN = 3
WCH = NCHUNK // NWIN
TAIL = E_W - NCHUNK * CHUNK
ROWS_T = 624
ROWS_TAIL = N - ROWS_T * NS
ZB = 800


def _make_sc_agg(compute_deg: bool):
    out_type = [jax.ShapeDtypeStruct((NC, N, D), jnp.float32)]
    if compute_deg:
        out_type.append(jax.ShapeDtypeStruct((NC, N), jnp.float32))

    WSZ = WCH * CHUNK
    scratch = [
        pltpu.VMEM((2, WSZ), jnp.int32),
        pltpu.VMEM((2, WSZ), jnp.int32),
        pltpu.VMEM((1, TAIL), jnp.int32),
        pltpu.VMEM((1, TAIL), jnp.int32),
        pltpu.VMEM((CHUNK, D), jnp.float32),
        pltpu.VMEM((CHUNK, D), jnp.float32),
        pltpu.VMEM((CHUNK,), jnp.float32),
        pltpu.VMEM((ZB,), jnp.float32),
        pltpu.VMEM_SHARED((N, D), jnp.float32),
        pltpu.VMEM_SHARED((N,), jnp.float32),
        pltpu.SemaphoreType.DMA,
        pltpu.SemaphoreType.DMA,
        pltpu.SemaphoreType.DMA,
    ]

    mesh = plsc.VectorSubcoreMesh(core_axis_name="c", subcore_axis_name="s")

    WSZ = WCH * CHUNK

    def body(x_hbm, e_hbm, *rest):
        if compute_deg:
            agg_out, deg_out = rest[0], rest[1]
            scr = rest[2:]
        else:
            agg_out = rest[0]
            deg_out = None
            scr = rest[1:]
        (src_v, dst_v, srct_v, dstt_v, rows0, rows1, ones_v, z1_v,
         agg_s, deg_s, sem0, sem1, semw) = scr

        c = lax.axis_index("c")
        s = lax.axis_index("s")
        wid = s * NC + c

        z16 = jnp.zeros((16,), jnp.float32)

        pltpu.sync_copy(e_hbm.at[0, wid, pl.ds(0, WSZ)], src_v.at[0])
        pltpu.sync_copy(e_hbm.at[1, wid, pl.ds(0, WSZ)], dst_v.at[0])
        pltpu.sync_copy(e_hbm.at[0, wid, pl.ds(NWIN * WSZ, TAIL)],
                        srct_v.at[0])
        pltpu.sync_copy(e_hbm.at[1, wid, pl.ds(NWIN * WSZ, TAIL)],
                        dstt_v.at[0])
        pltpu.async_copy(x_hbm.at[src_v.at[0, pl.ds(0, CHUNK)]], rows0, sem0)

        def zrow(i, carry):
            for j in range(D // 16):
                rows1[i, pl.ds(j * 16, 16)] = z16
            return carry

        lax.fori_loop(0, CHUNK, zrow, 0)

        r0 = s * ROWS_T
        n_full = ROWS_T // CHUNK
        rem = ROWS_T - n_full * CHUNK
        for t in range(n_full):
            pltpu.sync_copy(rows1, agg_s.at[pl.ds(r0 + t * CHUNK, CHUNK)])
        if rem:
            pltpu.sync_copy(rows1.at[pl.ds(0, rem)],
                            agg_s.at[pl.ds(r0 + n_full * CHUNK, rem)])

        @pl.when(s == 0)
        def _():
            pltpu.sync_copy(rows1.at[pl.ds(0, ROWS_TAIL)],
                            agg_s.at[pl.ds(ROWS_T * NS, ROWS_TAIL)])

        if compute_deg:
            def zz(i, carry):
                z1_v[pl.ds(i * 16, 16)] = z16
                return carry
            lax.fori_loop(0, ZB // 16, zz, 0)

            one16 = jnp.ones((16,), jnp.float32)
            for j in range(CHUNK // 16):
                ones_v[pl.ds(j * 16, 16)] = one16

            @pl.when(s == 0)
            def _():
                nf = N // ZB
                for t in range(nf):
                    pltpu.sync_copy(z1_v, deg_s.at[pl.ds(t * ZB, ZB)])
                drem = N - nf * ZB
                if drem:
                    pltpu.sync_copy(z1_v.at[pl.ds(0, drem)],
                                    deg_s.at[pl.ds(nf * ZB, drem)])

        plsc.subcore_barrier()

        def idx(v, b, k):
            return v.at[b, pl.ds(k * CHUNK, CHUNK)]

        def gather(b, k, buf, sem):
            return pltpu.async_copy(x_hbm.at[idx(src_v, b, k)], buf, sem)

        def scat(b, k, buf):
            pltpu.sync_copy(buf, agg_s.at[idx(dst_v, b, k)], add=True)
            if compute_deg:
                pltpu.sync_copy(ones_v, deg_s.at[idx(dst_v, b, k)], add=True)

        def make_step(b):
            def step(g, carry):
                k0 = 2 * g
                gather(b, k0 + 1, rows1, sem1)
                pltpu.make_async_copy(x_hbm.at[idx(src_v, b, k0)], rows0,
                                      sem0).wait()
                scat(b, k0, rows0)

                @pl.when(k0 + 2 < WCH)
                def _():
                    gather(b, k0 + 2, rows0, sem0)

                pltpu.make_async_copy(x_hbm.at[idx(src_v, b, k0 + 1)], rows1,
                                      sem1).wait()
                scat(b, k0 + 1, rows1)
                return carry
            return step

        for w in range(NWIN):
            b = w % 2
            if w + 1 < NWIN:
                nb = (w + 1) % 2
                pltpu.async_copy(e_hbm.at[0, wid, pl.ds((w + 1) * WSZ, WSZ)],
                                 src_v.at[nb], semw)
                pltpu.async_copy(e_hbm.at[1, wid, pl.ds((w + 1) * WSZ, WSZ)],
                                 dst_v.at[nb], semw)
            lax.fori_loop(0, WCH // 2, make_step(b), 0)
            if w + 1 < NWIN:
                nb = (w + 1) % 2
                pltpu.make_async_copy(
                    e_hbm.at[0, wid, pl.ds((w + 1) * WSZ, WSZ)],
                    src_v.at[nb], semw).wait()
                pltpu.make_async_copy(
                    e_hbm.at[1, wid, pl.ds((w + 1) * WSZ, WSZ)],
                    dst_v.at[nb], semw).wait()
                gather(nb, 0, rows0, sem0)

        pltpu.async_copy(x_hbm.at[srct_v.at[0]],
                         rows0.at[pl.ds(0, TAIL)], sem0).wait()
        pltpu.sync_copy(rows0.at[pl.ds(0, TAIL)],
                        agg_s.at[dstt_v.at[0]], add=True)
        if compute_deg:
            pltpu.sync_copy(ones_v.at[pl.ds(0, TAIL)],
                            deg_s.at[dstt_v.at[0]], add=True)

        plsc.subcore_barrier()

        pltpu.sync_copy(agg_s.at[pl.ds(r0, ROWS_T)],
                        agg_out.at[c, pl.ds(r0, ROWS_T)])

        @pl.when(s == 0)
        def _():
            pltpu.sync_copy(agg_s.at[pl.ds(ROWS_T * NS, ROWS_TAIL)],
                            agg_out.at[c, pl.ds(ROWS_T * NS, ROWS_TAIL)])

        if compute_deg:
            @pl.when(s == 0)
            def _():
                pltpu.sync_copy(deg_s, deg_out.at[c])

    return pl.kernel(body, out_type=out_type, scratch_types=scratch, mesh=mesh)


_sc_agg_deg = _make_sc_agg(True)
_sc_agg = _make_sc_agg(False)


RB = 2000
NB = N // RB


def _tc_hidden_body(aggp, degp, x, wl, bl, wr, o):
    agg = aggp[0] + aggp[1]
    deg = jnp.maximum(degp[0] + degp[1], 1.0)
    mean = agg / deg
    z = (lax.dot_general(mean, wl[...], (((1,), (1,)), ((), ())),
                         preferred_element_type=jnp.float32)
         + lax.dot_general(x[...], wr[...], (((1,), (1,)), ((), ())),
                           preferred_element_type=jnp.float32)
         + bl[...])
    o[...] = jnp.maximum(z, 0.0)


def _tc_final_body(aggp, degp, x, wl, bl, wr, o):
    agg = aggp[0] + aggp[1]
    deg = jnp.maximum(degp[0] + degp[1], 1.0)
    mean = agg / deg
    z = (lax.dot_general(mean, wl[...], (((1,), (1,)), ((), ())),
                         preferred_element_type=jnp.float32)
         + lax.dot_general(x[...], wr[...], (((1,), (1,)), ((), ())),
                           preferred_element_type=jnp.float32)
         + bl[...])
    m = jnp.max(z, axis=-1, keepdims=True)
    lse = jnp.log(jnp.sum(jnp.exp(z - m), axis=-1, keepdims=True)) + m
    o[...] = z - lse


def _tc_layer(body, aggp, degp, x, wl, bl, wr):
    return pl.pallas_call(
        body,
        grid=(NB,),
        in_specs=[
            pl.BlockSpec((NC, RB, D), lambda i: (0, i, 0)),
            pl.BlockSpec((NC, RB, 1), lambda i: (0, i, 0)),
            pl.BlockSpec((RB, D), lambda i: (i, 0)),
            pl.BlockSpec((D, D), lambda i: (0, 0)),
            pl.BlockSpec((1, D), lambda i: (0, 0)),
            pl.BlockSpec((D, D), lambda i: (0, 0)),
        ],
        out_specs=pl.BlockSpec((RB, D), lambda i: (i, 0)),
        out_shape=jax.ShapeDtypeStruct((N, D), jnp.float32),
    )(aggp, degp, x, wl, bl, wr)


def kernel(x, edge_index, Wl1, bl1, Wr1, Wl2, bl2, Wr2):
    ei = edge_index.astype(jnp.int32).reshape(2, NW, E_W)

    aggp1, degp = _sc_agg_deg(x, ei)
    degp3 = degp[:, :, None]
    h = _tc_layer(_tc_hidden_body, aggp1, degp3, x,
                  Wl1, bl1.reshape(1, D), Wr1)
    (aggp2,) = _sc_agg(h, ei)
    out = _tc_layer(_tc_final_body, aggp2, degp3, h,
                    Wl2, bl2.reshape(1, D), Wr2)
    return out


# speedup vs baseline: 13.2228x; 1.0124x over previous
import functools

import jax
import jax.numpy as jnp
from jax import lax
from jax.experimental import pallas as pl
from jax.experimental.pallas import tpu as pltpu
from jax.experimental.pallas import tpu_sc as plsc

N = 10000
E = 320000
D = 128

NC = 2
NS = 16
NW = NC * NS
CHUNK = 128
ECHUNKS = E // CHUNK
NCHUNK = 78
NEXTRA = ECHUNKS - NCHUNK * NW
NWIN = 3
WCH = NCHUNK // NWIN
WSZ = WCH * CHUNK
ROWS_T = 624
ROWS_TAIL = N - ROWS_T * NS
ZB = 800


def _make_sc_agg(compute_deg: bool):
    out_type = [jax.ShapeDtypeStruct((NC, N, D), jnp.float32)]
    if compute_deg:
        out_type.append(jax.ShapeDtypeStruct((NC, N), jnp.float32))

    scratch = [
        pltpu.VMEM((2, WSZ), jnp.int32),
        pltpu.VMEM((2, WSZ), jnp.int32),
        pltpu.VMEM((1, CHUNK), jnp.int32),
        pltpu.VMEM((1, CHUNK), jnp.int32),
        pltpu.VMEM((CHUNK, D), jnp.float32),
        pltpu.VMEM((CHUNK, D), jnp.float32),
        pltpu.VMEM((CHUNK,), jnp.float32),
        pltpu.VMEM((ZB,), jnp.float32),
        pltpu.VMEM_SHARED((N, D), jnp.float32),
        pltpu.VMEM_SHARED((N,), jnp.float32),
        pltpu.SemaphoreType.DMA,
        pltpu.SemaphoreType.DMA,
        pltpu.SemaphoreType.DMA,
    ]

    mesh = plsc.VectorSubcoreMesh(core_axis_name="c", subcore_axis_name="s")

    def body(x_hbm, e_hbm, *rest):
        if compute_deg:
            agg_out, deg_out = rest[0], rest[1]
            scr = rest[2:]
        else:
            agg_out = rest[0]
            deg_out = None
            scr = rest[1:]
        (src_v, dst_v, srcx_v, dstx_v, rows0, rows1, ones_v, z1_v,
         agg_s, deg_s, sem0, sem1, semw) = scr

        c = lax.axis_index("c")
        s = lax.axis_index("s")
        wid = s * NC + c
        eofs = pl.multiple_of(
            jnp.where(wid < NEXTRA, wid * (NCHUNK + 1),
                      NEXTRA * (NCHUNK + 1)
                      + (wid - NEXTRA) * NCHUNK) * CHUNK,
            CHUNK)

        z16 = jnp.zeros((16,), jnp.float32)

        pltpu.sync_copy(e_hbm.at[0, pl.ds(eofs, WSZ)], src_v.at[0])
        pltpu.sync_copy(e_hbm.at[1, pl.ds(eofs, WSZ)], dst_v.at[0])
        pltpu.async_copy(x_hbm.at[src_v.at[0, pl.ds(0, CHUNK)]], rows0, sem0)

        def zrow(i, carry):
            for j in range(D // 16):
                rows1[i, pl.ds(j * 16, 16)] = z16
            return carry

        lax.fori_loop(0, CHUNK, zrow, 0)

        r0 = s * ROWS_T
        n_full = ROWS_T // CHUNK
        rem = ROWS_T - n_full * CHUNK
        for t in range(n_full):
            pltpu.sync_copy(rows1, agg_s.at[pl.ds(r0 + t * CHUNK, CHUNK)])
        if rem:
            pltpu.sync_copy(rows1.at[pl.ds(0, rem)],
                            agg_s.at[pl.ds(r0 + n_full * CHUNK, rem)])

        @pl.when(s == 0)
        def _():
            pltpu.sync_copy(rows1.at[pl.ds(0, ROWS_TAIL)],
                            agg_s.at[pl.ds(ROWS_T * NS, ROWS_TAIL)])

        if compute_deg:
            def zz(i, carry):
                z1_v[pl.ds(i * 16, 16)] = z16
                return carry
            lax.fori_loop(0, ZB // 16, zz, 0)

            one16 = jnp.ones((16,), jnp.float32)
            for j in range(CHUNK // 16):
                ones_v[pl.ds(j * 16, 16)] = one16

            @pl.when(s == 0)
            def _():
                nf = N // ZB
                for t in range(nf):
                    pltpu.sync_copy(z1_v, deg_s.at[pl.ds(t * ZB, ZB)])
                drem = N - nf * ZB
                if drem:
                    pltpu.sync_copy(z1_v.at[pl.ds(0, drem)],
                                    deg_s.at[pl.ds(nf * ZB, drem)])

        plsc.subcore_barrier()

        def idx(v, b, k):
            return v.at[b, pl.ds(k * CHUNK, CHUNK)]

        def gather(b, k, buf, sem):
            return pltpu.async_copy(x_hbm.at[idx(src_v, b, k)], buf, sem)

        def scat(b, k, buf):
            pltpu.sync_copy(buf, agg_s.at[idx(dst_v, b, k)], add=True)
            if compute_deg:
                pltpu.sync_copy(ones_v, deg_s.at[idx(dst_v, b, k)], add=True)

        def make_step(b):
            def step(g, carry):
                k0 = 2 * g
                gather(b, k0 + 1, rows1, sem1)
                pltpu.make_async_copy(x_hbm.at[idx(src_v, b, k0)], rows0,
                                      sem0).wait()
                scat(b, k0, rows0)

                @pl.when(k0 + 2 < WCH)
                def _():
                    gather(b, k0 + 2, rows0, sem0)

                pltpu.make_async_copy(x_hbm.at[idx(src_v, b, k0 + 1)], rows1,
                                      sem1).wait()
                scat(b, k0 + 1, rows1)
                return carry
            return step

        for w in range(NWIN):
            b = w % 2
            if w + 1 < NWIN:
                nb = (w + 1) % 2
                wofs = eofs + (w + 1) * WSZ
                pltpu.async_copy(e_hbm.at[0, pl.ds(wofs, WSZ)],
                                 src_v.at[nb], semw)
                pltpu.async_copy(e_hbm.at[1, pl.ds(wofs, WSZ)],
                                 dst_v.at[nb], semw)
            lax.fori_loop(0, WCH // 2, make_step(b), 0)
            if w + 1 < NWIN:
                nb = (w + 1) % 2
                wofs = eofs + (w + 1) * WSZ
                pltpu.make_async_copy(e_hbm.at[0, pl.ds(wofs, WSZ)],
                                      src_v.at[nb], semw).wait()
                pltpu.make_async_copy(e_hbm.at[1, pl.ds(wofs, WSZ)],
                                      dst_v.at[nb], semw).wait()
                gather(nb, 0, rows0, sem0)

        @pl.when(wid < NEXTRA)
        def _():
            xofs = eofs + NCHUNK * CHUNK
            pltpu.sync_copy(e_hbm.at[0, pl.ds(xofs, CHUNK)], srcx_v.at[0])
            pltpu.sync_copy(e_hbm.at[1, pl.ds(xofs, CHUNK)], dstx_v.at[0])
            pltpu.async_copy(x_hbm.at[srcx_v.at[0]], rows0, sem0).wait()
            pltpu.sync_copy(rows0, agg_s.at[dstx_v.at[0]], add=True)
            if compute_deg:
                pltpu.sync_copy(ones_v, deg_s.at[dstx_v.at[0]], add=True)

        plsc.subcore_barrier()

        pltpu.sync_copy(agg_s.at[pl.ds(r0, ROWS_T)],
                        agg_out.at[c, pl.ds(r0, ROWS_T)])

        @pl.when(s == 0)
        def _():
            pltpu.sync_copy(agg_s.at[pl.ds(ROWS_T * NS, ROWS_TAIL)],
                            agg_out.at[c, pl.ds(ROWS_T * NS, ROWS_TAIL)])

        if compute_deg:
            @pl.when(s == 0)
            def _():
                pltpu.sync_copy(deg_s, deg_out.at[c])

    return pl.kernel(body, out_type=out_type, scratch_types=scratch, mesh=mesh)


_sc_agg_deg = _make_sc_agg(True)
_sc_agg = _make_sc_agg(False)


RB = 2000
NB = N // RB


def _combine(aggp, degp, x, wl, bl, wr):
    agg = aggp[0] + aggp[1]
    deg = jnp.maximum(degp[:, 0:1] + degp[:, 1:2], 1.0)
    mean = agg / deg
    return (lax.dot_general(mean, wl[...], (((1,), (1,)), ((), ())),
                            preferred_element_type=jnp.float32)
            + lax.dot_general(x[...], wr[...], (((1,), (1,)), ((), ())),
                              preferred_element_type=jnp.float32)
            + bl[...])


def _tc_hidden_body(aggp, degp, x, wl, bl, wr, o):
    z = _combine(aggp, degp[...], x, wl, bl, wr)
    o[...] = jnp.maximum(z, 0.0)


def _tc_final_body(aggp, degp, x, wl, bl, wr, o):
    z = _combine(aggp, degp[...], x, wl, bl, wr)
    m = jnp.max(z, axis=-1, keepdims=True)
    lse = jnp.log(jnp.sum(jnp.exp(z - m), axis=-1, keepdims=True)) + m
    o[...] = z - lse


def _tc_layer(body, aggp, degp, x, wl, bl, wr):
    return pl.pallas_call(
        body,
        grid=(NB,),
        in_specs=[
            pl.BlockSpec((NC, RB, D), lambda i: (0, i, 0)),
            pl.BlockSpec((RB, NC), lambda i: (i, 0)),
            pl.BlockSpec((RB, D), lambda i: (i, 0)),
            pl.BlockSpec((D, D), lambda i: (0, 0)),
            pl.BlockSpec((1, D), lambda i: (0, 0)),
            pl.BlockSpec((D, D), lambda i: (0, 0)),
        ],
        out_specs=pl.BlockSpec((RB, D), lambda i: (i, 0)),
        out_shape=jax.ShapeDtypeStruct((N, D), jnp.float32),
    )(aggp, degp, x, wl, bl, wr)


def kernel(x, edge_index, Wl1, bl1, Wr1, Wl2, bl2, Wr2):
    ei = edge_index.astype(jnp.int32)

    aggp1, degp = _sc_agg_deg(x, ei)
    degpt = degp.T
    h = _tc_layer(_tc_hidden_body, aggp1, degpt, x,
                  Wl1, bl1.reshape(1, D), Wr1)
    (aggp2,) = _sc_agg(h, ei)
    out = _tc_layer(_tc_final_body, aggp2, degpt, h,
                    Wl2, bl2.reshape(1, D), Wr2)
    return out
